# baseline (speedup 1.0000x reference)
import jax
import jax.numpy as jnp
import numpy as np
from jax import lax
from jax.experimental import pallas as pl
from jax.experimental.pallas import tpu as pltpu
from jax.experimental.pallas import tpu_sc as plsc

N = 10000
E = 320000
D = 128
DE = 16

_NC = 2
_NS = 16
_L = 16
_NW = _NC * _NS

_C = 40
_NCHUNK = E // _C
_NWCH = _NCHUNK // _NW
_SCH = 5
_CC = 128
_NCCHUNK = E // _CC
_TILE_ROWS = 640
_LAST_ROWS = N - 15 * _TILE_ROWS
_NPADC = 10240
_CPT = _NPADC // _NS
_BN = 1000
_BE = 2000


def _tile_rows(s):
    row0 = s * _TILE_ROWS
    return row0


_EW_SCALE = 2048.0

def _ew_body(ea_ref, wlo_ref, whi_ref, blo_ref, bhi_ref, o_ref):
    a = ea_ref[...]
    mlo = jnp.dot(a, wlo_ref[...], preferred_element_type=jnp.float32,
                  precision=lax.Precision.HIGHEST) + blo_ref[...]
    mhi = jnp.dot(a, whi_ref[...], preferred_element_type=jnp.float32,
                  precision=lax.Precision.HIGHEST) + bhi_ref[...]
    qlo = jnp.clip(jnp.round(mlo * _EW_SCALE), -32767.0, 32767.0).astype(jnp.int32)
    qhi = jnp.clip(jnp.round(mhi * _EW_SCALE), -32767.0, 32767.0).astype(jnp.int32)
    o_ref[...] = jnp.bitwise_or(jnp.bitwise_and(qlo, 0xFFFF),
                                jnp.left_shift(qhi, 16))


def _ew_call(ea, weT, b):
    grid = (E // _BE,)
    return pl.pallas_call(
        _ew_body,
        grid=grid,
        in_specs=[
            pl.BlockSpec((_BE, DE), lambda i: (i, 0)),
            pl.BlockSpec((DE, D // 2), lambda i: (0, 0)),
            pl.BlockSpec((DE, D // 2), lambda i: (0, 0)),
            pl.BlockSpec((1, D // 2), lambda i: (0, 0)),
            pl.BlockSpec((1, D // 2), lambda i: (0, 0)),
        ],
        out_specs=pl.BlockSpec((_BE, D // 2), lambda i: (i, 0)),
        out_shape=jax.ShapeDtypeStruct((E, D // 2), jnp.int32),
    )(ea, weT[:, :D // 2], weT[:, D // 2:],
      b[:D // 2].reshape(1, D // 2), b[D // 2:].reshape(1, D // 2))


def _xw_body(x_ref, w_ref, o_ref):
    o_ref[...] = jnp.dot(x_ref[...], w_ref[...], preferred_element_type=jnp.float32)


def _xw_call(x, wxT):
    grid = (N // _BN,)
    return pl.pallas_call(
        _xw_body,
        grid=grid,
        in_specs=[
            pl.BlockSpec((_BN, D), lambda i: (i, 0)),
            pl.BlockSpec((D, D), lambda i: (0, 0)),
        ],
        out_specs=pl.BlockSpec((_BN, D), lambda i: (i, 0)),
        out_shape=jax.ShapeDtypeStruct((N, D), jnp.float32),
    )(x, wxT)


def _update_body(x_ref, s_ref, r_ref, ux_ref, ua_ref, h_ref):
    agg = (s_ref[0] + s_ref[1]) * r_ref[...]
    h = jnp.dot(x_ref[...], ux_ref[...], preferred_element_type=jnp.float32)
    h += jnp.dot(agg, ua_ref[...], preferred_element_type=jnp.float32)
    h_ref[...] = jnp.maximum(h, 0.0)


def _update_call(x, s, recip, uxT, uaT):
    grid = (N // _BN,)
    return pl.pallas_call(
        _update_body,
        grid=grid,
        in_specs=[
            pl.BlockSpec((_BN, D), lambda i: (i, 0)),
            pl.BlockSpec((2, _BN, D), lambda i: (0, i, 0)),
            pl.BlockSpec((_BN, 1), lambda i: (i, 0)),
            pl.BlockSpec((D, D), lambda i: (0, 0)),
            pl.BlockSpec((D, D), lambda i: (0, 0)),
        ],
        out_specs=pl.BlockSpec((_BN, D), lambda i: (i, 0)),
        out_shape=jax.ShapeDtypeStruct((N, D), jnp.float32),
    )(x, s, recip, uxT, uaT)


def _update_xw_body(x_ref, s_ref, r_ref, ux_ref, ua_ref, wxn_ref, h_ref, xwn_ref):
    agg = (s_ref[0] + s_ref[1]) * r_ref[...]
    h = jnp.dot(x_ref[...], ux_ref[...], preferred_element_type=jnp.float32)
    h += jnp.dot(agg, ua_ref[...], preferred_element_type=jnp.float32)
    h = jnp.maximum(h, 0.0)
    h_ref[...] = h
    xwn_ref[...] = jnp.dot(h, wxn_ref[...], preferred_element_type=jnp.float32)


def _update_xw_call(x, s, recip, uxT, uaT, wxnT):
    grid = (N // _BN,)
    return pl.pallas_call(
        _update_xw_body,
        grid=grid,
        in_specs=[
            pl.BlockSpec((_BN, D), lambda i: (i, 0)),
            pl.BlockSpec((2, _BN, D), lambda i: (0, i, 0)),
            pl.BlockSpec((_BN, 1), lambda i: (i, 0)),
            pl.BlockSpec((D, D), lambda i: (0, 0)),
            pl.BlockSpec((D, D), lambda i: (0, 0)),
            pl.BlockSpec((D, D), lambda i: (0, 0)),
        ],
        out_specs=[
            pl.BlockSpec((_BN, D), lambda i: (i, 0)),
            pl.BlockSpec((_BN, D), lambda i: (i, 0)),
        ],
        out_shape=[
            jax.ShapeDtypeStruct((N, D), jnp.float32),
            jax.ShapeDtypeStruct((N, D), jnp.float32),
        ],
    )(x, s, recip, uxT, uaT, wxnT)


def _sc_mesh():
    return plsc.VectorSubcoreMesh(core_axis_name="c", subcore_axis_name="s",
                                  num_cores=_NC, num_subcores=_NS)


def _msg_body(xw_hbm, ewb_hbm, src_hbm, dst_hbm,
              sums_hbm,
              isa, ida, isb, idb,
              gx, ge,
              acc,
              sg0, sg1, se0, se1, ss0, ss1):
    c = lax.axis_index("c")
    s = lax.axis_index("s")
    wid = s * _NC + c

    zero16 = jnp.zeros((_L,), jnp.float32)

    def _zero_row(i, _):
        for j in range(D // _L):
            gx[0, i, pl.ds(j * _L, _L)] = zero16
        return 0

    lax.fori_loop(0, _C, _zero_row, 0)

    row0 = s * _TILE_ROWS
    for k in range(-(-_TILE_ROWS // _C)):
        ln = min(_C, _TILE_ROWS - k * _C)

        @pl.when(jnp.logical_or(s < _NS - 1, k * _C + ln <= _LAST_ROWS))
        def _():
            pltpu.sync_copy(gx.at[0, pl.ds(0, ln)],
                            acc.at[pl.ds(row0 + k * _C, ln)])

    @pl.when(s == _NS - 1)
    def _():
        tail = _LAST_ROWS % _C
        if tail:
            base = 15 * _TILE_ROWS + (_LAST_ROWS // _C) * _C
            pltpu.sync_copy(gx.at[0, pl.ds(0, tail)], acc.at[pl.ds(base, tail)])

    plsc.subcore_barrier()

    base_chunk = wid * _NWCH
    gsem = (sg0, sg1)
    esem = (se0, se1)
    ssem = (ss0, ss1)
    sbuf = (isa, isb)
    dbuf = (ida, idb)

    def _load_idx(t1, kb):
        pltpu.sync_copy(src_hbm.at[pl.ds(base_chunk + t1, _SCH)], sbuf[kb])
        pltpu.sync_copy(dst_hbm.at[pl.ds(base_chunk + t1, _SCH)], dbuf[kb])

    def _issue(t1, kb, slot, st):
        pltpu.async_copy(xw_hbm.at[sbuf[kb].at[slot, 0]], gx.at[st], gsem[st])
        pltpu.async_copy(ewb_hbm.at[pl.ds((base_chunk + t1) * _C, _C)],
                         ge.at[st], esem[st])

    def _wait(t1, kb, slot, st):
        pltpu.make_async_copy(xw_hbm.at[sbuf[kb].at[slot, 0]], gx.at[st],
                              gsem[st]).wait()
        pltpu.make_async_copy(ewb_hbm.at[pl.ds((base_chunk + t1) * _C, _C)],
                              ge.at[st], esem[st]).wait()

    def _compute(st):
        inv = 1.0 / _EW_SCALE

        def _rows(i4, _):
            for r in range(4):
                i = i4 * 4 + r
                for g in range(D // 32):
                    w = ge[st, i, pl.ds(g * _L, _L)]
                    lo = lax.shift_right_arithmetic(lax.shift_left(w, 16), 16)
                    hi = lax.shift_right_arithmetic(w, 16)
                    lof = lo.astype(jnp.float32) * inv
                    hif = hi.astype(jnp.float32) * inv
                    sl0 = pl.ds(g * 32, _L)
                    sl1 = pl.ds(g * 32 + _L, _L)
                    gx[st, i, sl0] = jnp.maximum(gx[st, i, sl0] + lof, 0.0)
                    gx[st, i, sl1] = jnp.maximum(gx[st, i, sl1] + hif, 0.0)
            return 0

        lax.fori_loop(0, _C // 4, _rows, 0)

    def _scatter(st, kb, slot):
        pltpu.async_copy(gx.at[st], acc.at[dbuf[kb].at[slot, 0]], ssem[st],
                         add=True)

    def _wait_scatter(st, kb, slot):
        pltpu.make_async_copy(gx.at[st], acc.at[dbuf[kb].at[slot, 0]],
                              ssem[st]).wait()

    _load_idx(0, 0)
    _issue(0, 0, 0, 0)

    def _outer(t10, _):
        for k in range(10):
            t = t10 * 10 + k
            st = k % 2
            kb = (k // 5) % 2
            slot = k % 5
            nkb = ((k + 1) // 5) % 2
            nslot = (k + 1) % 5
            nst = (k + 1) % 2
            pkb = ((k - 1) // 5) % 2 if k >= 1 else ((9 // 5) % 2 if True else 0)
            pslot = (k - 1) % 5
            valid = t < _NWCH
            nvalid = t + 1 < _NWCH

            @pl.when(jnp.logical_and(t >= 1, valid))
            def _():
                _wait_scatter(nst, pkb, pslot)

            if (k + 1) % _SCH == 0:
                @pl.when(nvalid)
                def _():
                    _load_idx(t + 1, nkb)

            @pl.when(nvalid)
            def _():
                _issue(t + 1, nkb, nslot, nst)

            @pl.when(valid)
            def _():
                _wait(t, kb, slot, st)
                _compute(st)
                _scatter(st, kb, slot)
        return 0

    lax.fori_loop(0, -(-_NWCH // 10), _outer, 0)

    _wait_scatter((_NWCH - 1) % 2, ((_NWCH - 1) // 5) % 2, (_NWCH - 1) % 5)

    plsc.subcore_barrier()

    @pl.when(s < _NS - 1)
    def _():
        pltpu.sync_copy(acc.at[pl.ds(row0, _TILE_ROWS)],
                        sums_hbm.at[c, pl.ds(row0, _TILE_ROWS)])

    @pl.when(s == _NS - 1)
    def _():
        pltpu.sync_copy(acc.at[pl.ds(15 * _TILE_ROWS, _LAST_ROWS)],
                        sums_hbm.at[c, pl.ds(15 * _TILE_ROWS, _LAST_ROWS)])


def _msg_call(xw, ewb, src2, dst2):
    f = pl.kernel(
        _msg_body,
        out_type=jax.ShapeDtypeStruct((_NC, N, D), jnp.float32),
        mesh=_sc_mesh(),
        scratch_types=[
            pltpu.VMEM((_SCH, 1, _C), jnp.int32),
            pltpu.VMEM((_SCH, 1, _C), jnp.int32),
            pltpu.VMEM((_SCH, 1, _C), jnp.int32),
            pltpu.VMEM((_SCH, 1, _C), jnp.int32),
            pltpu.VMEM((2, _C, D), jnp.float32),
            pltpu.VMEM((2, _C, D // 2), jnp.int32),
            pltpu.VMEM_SHARED((N, D), jnp.float32),
            pltpu.SemaphoreType.DMA,
            pltpu.SemaphoreType.DMA,
            pltpu.SemaphoreType.DMA,
            pltpu.SemaphoreType.DMA,
            pltpu.SemaphoreType.DMA,
            pltpu.SemaphoreType.DMA,
        ],
    )
    return f(xw, ewb, src2, dst2)


def _cnt_body(dst_hbm,
              cnts_hbm,
              ida, idb, ones_v,
              accc,
              ss0, ss1):
    c = lax.axis_index("c")
    s = lax.axis_index("s")
    wid = s * _NC + c

    zero16 = jnp.zeros((_L,), jnp.float32)
    one16 = jnp.ones((_L,), jnp.float32)

    def _fill(val):
        def _row(i, _):
            for j in range(D // _L):
                ones_v[i, pl.ds(j * _L, _L)] = val
            return 0
        lax.fori_loop(0, _CC, _row, 0)

    _fill(zero16)

    row0 = s * _TILE_ROWS
    for k in range(_TILE_ROWS // _CC):
        @pl.when(jnp.logical_or(s < _NS - 1, (k + 1) * _CC <= _LAST_ROWS))
        def _():
            pltpu.sync_copy(ones_v.at[pl.ds(0, _CC)],
                            accc.at[pl.ds(row0 + k * _CC, _CC)])

    @pl.when(s == _NS - 1)
    def _():
        tail = _LAST_ROWS % _CC
        if tail:
            base = 15 * _TILE_ROWS + (_LAST_ROWS // _CC) * _CC
            pltpu.sync_copy(ones_v.at[pl.ds(0, tail)], accc.at[pl.ds(base, tail)])

    _fill(one16)
    plsc.subcore_barrier()

    n_w = jnp.where(wid < 4, 79, 78)
    base_chunk = wid * 78 + jnp.minimum(wid, 4)
    dbuf = (ida, idb)
    ssem = (ss0, ss1)

    pltpu.sync_copy(dst_hbm.at[pl.ds(base_chunk, 1)], dbuf[0])

    def _outer(t2, _):
        for k in range(2):
            t = t2 * 2 + k
            st = k
            nst = 1 - k

            @pl.when(jnp.logical_and(t >= 1, t <= n_w))
            def _():
                pltpu.make_async_copy(ones_v, accc.at[dbuf[nst].at[0, 0]],
                                      ssem[nst]).wait()

            @pl.when(t + 1 < n_w)
            def _():
                pltpu.sync_copy(dst_hbm.at[pl.ds(base_chunk + t + 1, 1)],
                                dbuf[nst])

            @pl.when(t < n_w)
            def _():
                pltpu.async_copy(ones_v, accc.at[dbuf[st].at[0, 0]], ssem[st],
                                 add=True)
        return 0

    lax.fori_loop(0, 40, _outer, 0)

    plsc.subcore_barrier()

    @pl.when(s < _NS - 1)
    def _():
        pltpu.sync_copy(accc.at[pl.ds(row0, _TILE_ROWS)],
                        cnts_hbm.at[c, pl.ds(row0, _TILE_ROWS)])

    @pl.when(s == _NS - 1)
    def _():
        pltpu.sync_copy(accc.at[pl.ds(15 * _TILE_ROWS, _LAST_ROWS)],
                        cnts_hbm.at[c, pl.ds(15 * _TILE_ROWS, _LAST_ROWS)])


def _cnt_call(dst):
    f = pl.kernel(
        _cnt_body,
        out_type=jax.ShapeDtypeStruct((_NC, N, D), jnp.float32),
        mesh=_sc_mesh(),
        scratch_types=[
            pltpu.VMEM((1, 1, _CC), jnp.int32),
            pltpu.VMEM((1, 1, _CC), jnp.int32),
            pltpu.VMEM((_CC, D), jnp.float32),
            pltpu.VMEM_SHARED((N, D), jnp.float32),
            pltpu.SemaphoreType.DMA,
            pltpu.SemaphoreType.DMA,
        ],
    )
    return f(dst)


def kernel(x, edge_index, edge_attr, W1, b1, U1, W2, b2, U2):
    src = edge_index[0].astype(jnp.int32).reshape(_NCHUNK, 1, _C)
    dst_flat = edge_index[1].astype(jnp.int32)
    dst = dst_flat.reshape(_NCHUNK, 1, _C)

    pi = np.empty(D, np.int32)
    for j in range(D):
        if j < D // 2:
            pi[j] = 32 * (j // _L) + j % _L
        else:
            jj = j - D // 2
            pi[j] = 32 * (jj // _L) + _L + jj % _L
    w1xT = W1[:, :D].T
    w1eT = W1[:, D:].T[:, pi]
    u1xT = U1[:, :D].T
    u1aT = U1[:, D:].T
    w2xT = W2[:, :D].T
    w2eT = W2[:, D:].T[:, pi]
    u2xT = U2[:, :D].T
    u2aT = U2[:, D:].T

    cnt = _cnt_call(dst_flat.reshape(_NCCHUNK, 1, _CC))[:, :, 0]
    b1p = b1[pi]
    ewb1 = _ew_call(edge_attr, w1eT, b1p)
    xw1 = _xw_call(x, w1xT)
    recip = (1.0 / jnp.maximum(cnt[0] + cnt[1], 1.0))[:, None]

    s1 = _msg_call(xw1, ewb1, src, dst)
    ewb2 = _ew_call(edge_attr, w2eT, b2[pi])
    h1, xw2 = _update_xw_call(x, s1, recip, u1xT, u1aT, w2xT)

    s2 = _msg_call(xw2, ewb2, src, dst)
    h2 = _update_call(h1, s2, recip, u2xT, u2aT)
    return h2


# speedup vs baseline: 1.0547x; 1.0547x over previous
import jax
import jax.numpy as jnp
import numpy as np
from jax import lax
from jax.experimental import pallas as pl
from jax.experimental.pallas import tpu as pltpu
from jax.experimental.pallas import tpu_sc as plsc

N = 10000
E = 320000
D = 128
DE = 16

_NC = 2
_NS = 16
_L = 16
_NW = _NC * _NS

_C = 80
_NCHUNK = E // _C
_NWCH = _NCHUNK // _NW
_SCH = 5
_CC = 128
_NCCHUNK = E // _CC
_TILE_ROWS = 640
_LAST_ROWS = N - 15 * _TILE_ROWS
_NPADC = 10240
_CPT = _NPADC // _NS
_BN = 1000
_BE = 2000


def _tile_rows(s):
    row0 = s * _TILE_ROWS
    return row0


_EW_SCALE = 2048.0

def _ew_body(ea_ref, wlo_ref, whi_ref, blo_ref, bhi_ref, o_ref):
    a = ea_ref[...]
    mlo = jnp.dot(a, wlo_ref[...], preferred_element_type=jnp.float32,
                  precision=lax.Precision.HIGHEST) + blo_ref[...]
    mhi = jnp.dot(a, whi_ref[...], preferred_element_type=jnp.float32,
                  precision=lax.Precision.HIGHEST) + bhi_ref[...]
    qlo = jnp.clip(jnp.round(mlo * _EW_SCALE), -32767.0, 32767.0).astype(jnp.int32)
    qhi = jnp.clip(jnp.round(mhi * _EW_SCALE), -32767.0, 32767.0).astype(jnp.int32)
    o_ref[...] = jnp.bitwise_or(jnp.bitwise_and(qlo, 0xFFFF),
                                jnp.left_shift(qhi, 16))


def _ew_call(ea, weT, b):
    grid = (E // _BE,)
    return pl.pallas_call(
        _ew_body,
        grid=grid,
        in_specs=[
            pl.BlockSpec((_BE, DE), lambda i: (i, 0)),
            pl.BlockSpec((DE, D // 2), lambda i: (0, 0)),
            pl.BlockSpec((DE, D // 2), lambda i: (0, 0)),
            pl.BlockSpec((1, D // 2), lambda i: (0, 0)),
            pl.BlockSpec((1, D // 2), lambda i: (0, 0)),
        ],
        out_specs=pl.BlockSpec((_BE, D // 2), lambda i: (i, 0)),
        out_shape=jax.ShapeDtypeStruct((E, D // 2), jnp.int32),
    )(ea, weT[:, :D // 2], weT[:, D // 2:],
      b[:D // 2].reshape(1, D // 2), b[D // 2:].reshape(1, D // 2))


def _xw_body(x_ref, w_ref, o_ref):
    o_ref[...] = jnp.dot(x_ref[...], w_ref[...], preferred_element_type=jnp.float32)


def _xw_call(x, wxT):
    grid = (N // _BN,)
    return pl.pallas_call(
        _xw_body,
        grid=grid,
        in_specs=[
            pl.BlockSpec((_BN, D), lambda i: (i, 0)),
            pl.BlockSpec((D, D), lambda i: (0, 0)),
        ],
        out_specs=pl.BlockSpec((_BN, D), lambda i: (i, 0)),
        out_shape=jax.ShapeDtypeStruct((N, D), jnp.float32),
    )(x, wxT)


def _update_body(x_ref, s_ref, r_ref, ux_ref, ua_ref, h_ref):
    agg = (s_ref[0] + s_ref[1]) * r_ref[...]
    h = jnp.dot(x_ref[...], ux_ref[...], preferred_element_type=jnp.float32)
    h += jnp.dot(agg, ua_ref[...], preferred_element_type=jnp.float32)
    h_ref[...] = jnp.maximum(h, 0.0)


def _update_call(x, s, recip, uxT, uaT):
    grid = (N // _BN,)
    return pl.pallas_call(
        _update_body,
        grid=grid,
        in_specs=[
            pl.BlockSpec((_BN, D), lambda i: (i, 0)),
            pl.BlockSpec((2, _BN, D), lambda i: (0, i, 0)),
            pl.BlockSpec((_BN, 1), lambda i: (i, 0)),
            pl.BlockSpec((D, D), lambda i: (0, 0)),
            pl.BlockSpec((D, D), lambda i: (0, 0)),
        ],
        out_specs=pl.BlockSpec((_BN, D), lambda i: (i, 0)),
        out_shape=jax.ShapeDtypeStruct((N, D), jnp.float32),
    )(x, s, recip, uxT, uaT)


def _update_xw_body(x_ref, s_ref, r_ref, ux_ref, ua_ref, wxn_ref, h_ref, xwn_ref):
    agg = (s_ref[0] + s_ref[1]) * r_ref[...]
    h = jnp.dot(x_ref[...], ux_ref[...], preferred_element_type=jnp.float32)
    h += jnp.dot(agg, ua_ref[...], preferred_element_type=jnp.float32)
    h = jnp.maximum(h, 0.0)
    h_ref[...] = h
    xwn_ref[...] = jnp.dot(h, wxn_ref[...], preferred_element_type=jnp.float32)


def _update_xw_call(x, s, recip, uxT, uaT, wxnT):
    grid = (N // _BN,)
    return pl.pallas_call(
        _update_xw_body,
        grid=grid,
        in_specs=[
            pl.BlockSpec((_BN, D), lambda i: (i, 0)),
            pl.BlockSpec((2, _BN, D), lambda i: (0, i, 0)),
            pl.BlockSpec((_BN, 1), lambda i: (i, 0)),
            pl.BlockSpec((D, D), lambda i: (0, 0)),
            pl.BlockSpec((D, D), lambda i: (0, 0)),
            pl.BlockSpec((D, D), lambda i: (0, 0)),
        ],
        out_specs=[
            pl.BlockSpec((_BN, D), lambda i: (i, 0)),
            pl.BlockSpec((_BN, D), lambda i: (i, 0)),
        ],
        out_shape=[
            jax.ShapeDtypeStruct((N, D), jnp.float32),
            jax.ShapeDtypeStruct((N, D), jnp.float32),
        ],
    )(x, s, recip, uxT, uaT, wxnT)


def _sc_mesh():
    return plsc.VectorSubcoreMesh(core_axis_name="c", subcore_axis_name="s",
                                  num_cores=_NC, num_subcores=_NS)


def _msg_body(xw_hbm, ewb_hbm, src_hbm, dst_hbm,
              sums_hbm,
              isa, ida, isb, idb,
              gx, ge,
              acc,
              sg0, sg1, se0, se1, ss0, ss1):
    c = lax.axis_index("c")
    s = lax.axis_index("s")
    wid = s * _NC + c

    zero16 = jnp.zeros((_L,), jnp.float32)

    def _zero_row(i, _):
        for j in range(D // _L):
            gx[0, i, pl.ds(j * _L, _L)] = zero16
        return 0

    lax.fori_loop(0, _C, _zero_row, 0)

    row0 = s * _TILE_ROWS
    for k in range(-(-_TILE_ROWS // _C)):
        ln = min(_C, _TILE_ROWS - k * _C)

        @pl.when(jnp.logical_or(s < _NS - 1, k * _C + ln <= _LAST_ROWS))
        def _():
            pltpu.sync_copy(gx.at[0, pl.ds(0, ln)],
                            acc.at[pl.ds(row0 + k * _C, ln)])

    @pl.when(s == _NS - 1)
    def _():
        tail = _LAST_ROWS % _C
        if tail:
            base = 15 * _TILE_ROWS + (_LAST_ROWS // _C) * _C
            pltpu.sync_copy(gx.at[0, pl.ds(0, tail)], acc.at[pl.ds(base, tail)])

    plsc.subcore_barrier()

    base_chunk = wid * _NWCH
    gsem = (sg0, sg1)
    esem = (se0, se1)
    ssem = (ss0, ss1)
    sbuf = (isa, isb)
    dbuf = (ida, idb)

    def _load_idx(t1, kb):
        pltpu.sync_copy(src_hbm.at[pl.ds(base_chunk + t1, _SCH)], sbuf[kb])
        pltpu.sync_copy(dst_hbm.at[pl.ds(base_chunk + t1, _SCH)], dbuf[kb])

    def _issue(t1, kb, slot, st):
        pltpu.async_copy(xw_hbm.at[sbuf[kb].at[slot, 0]], gx.at[st], gsem[st])
        pltpu.async_copy(ewb_hbm.at[pl.ds((base_chunk + t1) * _C, _C)],
                         ge.at[st], esem[st])

    def _wait(t1, kb, slot, st):
        pltpu.make_async_copy(xw_hbm.at[sbuf[kb].at[slot, 0]], gx.at[st],
                              gsem[st]).wait()
        pltpu.make_async_copy(ewb_hbm.at[pl.ds((base_chunk + t1) * _C, _C)],
                              ge.at[st], esem[st]).wait()

    def _compute(st):
        inv = 1.0 / _EW_SCALE

        def _rows(i4, _):
            for r in range(4):
                i = i4 * 4 + r
                for g in range(D // 32):
                    w = ge[st, i, pl.ds(g * _L, _L)]
                    lo = lax.shift_right_arithmetic(lax.shift_left(w, 16), 16)
                    hi = lax.shift_right_arithmetic(w, 16)
                    lof = lo.astype(jnp.float32) * inv
                    hif = hi.astype(jnp.float32) * inv
                    sl0 = pl.ds(g * 32, _L)
                    sl1 = pl.ds(g * 32 + _L, _L)
                    gx[st, i, sl0] = jnp.maximum(gx[st, i, sl0] + lof, 0.0)
                    gx[st, i, sl1] = jnp.maximum(gx[st, i, sl1] + hif, 0.0)
            return 0

        lax.fori_loop(0, _C // 4, _rows, 0)

    def _scatter(st, kb, slot):
        pltpu.async_copy(gx.at[st], acc.at[dbuf[kb].at[slot, 0]], ssem[st],
                         add=True)

    def _wait_scatter(st, kb, slot):
        pltpu.make_async_copy(gx.at[st], acc.at[dbuf[kb].at[slot, 0]],
                              ssem[st]).wait()

    _load_idx(0, 0)
    _issue(0, 0, 0, 0)

    def _outer(t10, _):
        for k in range(10):
            t = t10 * 10 + k
            st = k % 2
            kb = (k // 5) % 2
            slot = k % 5
            nkb = ((k + 1) // 5) % 2
            nslot = (k + 1) % 5
            nst = (k + 1) % 2
            pkb = ((k - 1) // 5) % 2 if k >= 1 else ((9 // 5) % 2 if True else 0)
            pslot = (k - 1) % 5
            valid = t < _NWCH
            nvalid = t + 1 < _NWCH

            @pl.when(jnp.logical_and(t >= 1, valid))
            def _():
                _wait_scatter(nst, pkb, pslot)

            if (k + 1) % _SCH == 0:
                @pl.when(nvalid)
                def _():
                    _load_idx(t + 1, nkb)

            @pl.when(nvalid)
            def _():
                _issue(t + 1, nkb, nslot, nst)

            @pl.when(valid)
            def _():
                _wait(t, kb, slot, st)
                _compute(st)
                _scatter(st, kb, slot)
        return 0

    lax.fori_loop(0, -(-_NWCH // 10), _outer, 0)

    _wait_scatter((_NWCH - 1) % 2, ((_NWCH - 1) // 5) % 2, (_NWCH - 1) % 5)

    plsc.subcore_barrier()

    @pl.when(s < _NS - 1)
    def _():
        pltpu.sync_copy(acc.at[pl.ds(row0, _TILE_ROWS)],
                        sums_hbm.at[c, pl.ds(row0, _TILE_ROWS)])

    @pl.when(s == _NS - 1)
    def _():
        pltpu.sync_copy(acc.at[pl.ds(15 * _TILE_ROWS, _LAST_ROWS)],
                        sums_hbm.at[c, pl.ds(15 * _TILE_ROWS, _LAST_ROWS)])


def _msg_call(xw, ewb, src2, dst2):
    f = pl.kernel(
        _msg_body,
        out_type=jax.ShapeDtypeStruct((_NC, N, D), jnp.float32),
        mesh=_sc_mesh(),
        scratch_types=[
            pltpu.VMEM((_SCH, 1, _C), jnp.int32),
            pltpu.VMEM((_SCH, 1, _C), jnp.int32),
            pltpu.VMEM((_SCH, 1, _C), jnp.int32),
            pltpu.VMEM((_SCH, 1, _C), jnp.int32),
            pltpu.VMEM((2, _C, D), jnp.float32),
            pltpu.VMEM((2, _C, D // 2), jnp.int32),
            pltpu.VMEM_SHARED((N, D), jnp.float32),
            pltpu.SemaphoreType.DMA,
            pltpu.SemaphoreType.DMA,
            pltpu.SemaphoreType.DMA,
            pltpu.SemaphoreType.DMA,
            pltpu.SemaphoreType.DMA,
            pltpu.SemaphoreType.DMA,
        ],
    )
    return f(xw, ewb, src2, dst2)


def _cnt_body(dst_hbm,
              cnts_hbm,
              ida, idb, ones_v,
              accc,
              ss0, ss1):
    c = lax.axis_index("c")
    s = lax.axis_index("s")
    wid = s * _NC + c

    zero16 = jnp.zeros((_L,), jnp.float32)
    one16 = jnp.ones((_L,), jnp.float32)

    def _fill(val):
        def _row(i, _):
            for j in range(D // _L):
                ones_v[i, pl.ds(j * _L, _L)] = val
            return 0
        lax.fori_loop(0, _CC, _row, 0)

    _fill(zero16)

    row0 = s * _TILE_ROWS
    for k in range(_TILE_ROWS // _CC):
        @pl.when(jnp.logical_or(s < _NS - 1, (k + 1) * _CC <= _LAST_ROWS))
        def _():
            pltpu.sync_copy(ones_v.at[pl.ds(0, _CC)],
                            accc.at[pl.ds(row0 + k * _CC, _CC)])

    @pl.when(s == _NS - 1)
    def _():
        tail = _LAST_ROWS % _CC
        if tail:
            base = 15 * _TILE_ROWS + (_LAST_ROWS // _CC) * _CC
            pltpu.sync_copy(ones_v.at[pl.ds(0, tail)], accc.at[pl.ds(base, tail)])

    _fill(one16)
    plsc.subcore_barrier()

    n_w = jnp.where(wid < 4, 79, 78)
    base_chunk = wid * 78 + jnp.minimum(wid, 4)
    dbuf = (ida, idb)
    ssem = (ss0, ss1)

    pltpu.sync_copy(dst_hbm.at[pl.ds(base_chunk, 1)], dbuf[0])

    def _outer(t2, _):
        for k in range(2):
            t = t2 * 2 + k
            st = k
            nst = 1 - k

            @pl.when(jnp.logical_and(t >= 1, t <= n_w))
            def _():
                pltpu.make_async_copy(ones_v, accc.at[dbuf[nst].at[0, 0]],
                                      ssem[nst]).wait()

            @pl.when(t + 1 < n_w)
            def _():
                pltpu.sync_copy(dst_hbm.at[pl.ds(base_chunk + t + 1, 1)],
                                dbuf[nst])

            @pl.when(t < n_w)
            def _():
                pltpu.async_copy(ones_v, accc.at[dbuf[st].at[0, 0]], ssem[st],
                                 add=True)
        return 0

    lax.fori_loop(0, 40, _outer, 0)

    plsc.subcore_barrier()

    @pl.when(s < _NS - 1)
    def _():
        pltpu.sync_copy(accc.at[pl.ds(row0, _TILE_ROWS)],
                        cnts_hbm.at[c, pl.ds(row0, _TILE_ROWS)])

    @pl.when(s == _NS - 1)
    def _():
        pltpu.sync_copy(accc.at[pl.ds(15 * _TILE_ROWS, _LAST_ROWS)],
                        cnts_hbm.at[c, pl.ds(15 * _TILE_ROWS, _LAST_ROWS)])


def _cnt_call(dst):
    f = pl.kernel(
        _cnt_body,
        out_type=jax.ShapeDtypeStruct((_NC, N, D), jnp.float32),
        mesh=_sc_mesh(),
        scratch_types=[
            pltpu.VMEM((1, 1, _CC), jnp.int32),
            pltpu.VMEM((1, 1, _CC), jnp.int32),
            pltpu.VMEM((_CC, D), jnp.float32),
            pltpu.VMEM_SHARED((N, D), jnp.float32),
            pltpu.SemaphoreType.DMA,
            pltpu.SemaphoreType.DMA,
        ],
    )
    return f(dst)


def kernel(x, edge_index, edge_attr, W1, b1, U1, W2, b2, U2):
    src = edge_index[0].astype(jnp.int32).reshape(_NCHUNK, 1, _C)
    dst_flat = edge_index[1].astype(jnp.int32)
    dst = dst_flat.reshape(_NCHUNK, 1, _C)

    pi = np.empty(D, np.int32)
    for j in range(D):
        if j < D // 2:
            pi[j] = 32 * (j // _L) + j % _L
        else:
            jj = j - D // 2
            pi[j] = 32 * (jj // _L) + _L + jj % _L
    w1xT = W1[:, :D].T
    w1eT = W1[:, D:].T[:, pi]
    u1xT = U1[:, :D].T
    u1aT = U1[:, D:].T
    w2xT = W2[:, :D].T
    w2eT = W2[:, D:].T[:, pi]
    u2xT = U2[:, :D].T
    u2aT = U2[:, D:].T

    cnt = _cnt_call(dst_flat.reshape(_NCCHUNK, 1, _CC))[:, :, 0]
    b1p = b1[pi]
    ewb1 = _ew_call(edge_attr, w1eT, b1p)
    xw1 = _xw_call(x, w1xT)
    recip = (1.0 / jnp.maximum(cnt[0] + cnt[1], 1.0))[:, None]

    s1 = _msg_call(xw1, ewb1, src, dst)
    ewb2 = _ew_call(edge_attr, w2eT, b2[pi])
    h1, xw2 = _update_xw_call(x, s1, recip, u1xT, u1aT, w2xT)

    s2 = _msg_call(xw2, ewb2, src, dst)
    h2 = _update_call(h1, s2, recip, u2xT, u2aT)
    return h2


# speedup vs baseline: 1.1576x; 1.0976x over previous
import jax
import jax.numpy as jnp
from jax import lax
from jax.experimental import pallas as pl
from jax.experimental.pallas import tpu as pltpu
from jax.experimental.pallas import tpu_sc as plsc

N = 10000
E = 320000
D = 128
DE = 16

_NC = 2
_NS = 16
_L = 16
_NW = _NC * _NS

_C = 40
_NCHUNK = E // _C
_NWCH = _NCHUNK // _NW
_SCH = 5
_CC = 128
_NCCHUNK = E // _CC
_TILE_ROWS = 640
_LAST_ROWS = N - 15 * _TILE_ROWS
_NPADC = 10240
_CPT = _NPADC // _NS
_BN = 1000
_BE = 2000


def _tile_rows(s):
    row0 = s * _TILE_ROWS
    return row0


def _ew_body(ea_ref, w_ref, b_ref, o_ref):
    a = ea_ref[...]
    o_ref[...] = jnp.dot(a, w_ref[...], preferred_element_type=jnp.float32) + b_ref[...]


def _ew_call(ea, weT, b):
    grid = (E // _BE,)
    return pl.pallas_call(
        _ew_body,
        grid=grid,
        in_specs=[
            pl.BlockSpec((_BE, DE), lambda i: (i, 0)),
            pl.BlockSpec((DE, D), lambda i: (0, 0)),
            pl.BlockSpec((1, D), lambda i: (0, 0)),
        ],
        out_specs=pl.BlockSpec((_BE, D), lambda i: (i, 0)),
        out_shape=jax.ShapeDtypeStruct((E, D), jnp.float32),
    )(ea, weT, b.reshape(1, D))


def _xw_body(x_ref, w_ref, o_ref):
    o_ref[...] = jnp.dot(x_ref[...], w_ref[...], preferred_element_type=jnp.float32)


def _xw_call(x, wxT):
    grid = (N // _BN,)
    return pl.pallas_call(
        _xw_body,
        grid=grid,
        in_specs=[
            pl.BlockSpec((_BN, D), lambda i: (i, 0)),
            pl.BlockSpec((D, D), lambda i: (0, 0)),
        ],
        out_specs=pl.BlockSpec((_BN, D), lambda i: (i, 0)),
        out_shape=jax.ShapeDtypeStruct((N, D), jnp.float32),
    )(x, wxT)


def _update_body(x_ref, s_ref, r_ref, ux_ref, ua_ref, h_ref):
    agg = (s_ref[0] + s_ref[1]) * r_ref[...]
    h = jnp.dot(x_ref[...], ux_ref[...], preferred_element_type=jnp.float32)
    h += jnp.dot(agg, ua_ref[...], preferred_element_type=jnp.float32)
    h_ref[...] = jnp.maximum(h, 0.0)


def _update_call(x, s, recip, uxT, uaT):
    grid = (N // _BN,)
    return pl.pallas_call(
        _update_body,
        grid=grid,
        in_specs=[
            pl.BlockSpec((_BN, D), lambda i: (i, 0)),
            pl.BlockSpec((2, _BN, D), lambda i: (0, i, 0)),
            pl.BlockSpec((_BN, 1), lambda i: (i, 0)),
            pl.BlockSpec((D, D), lambda i: (0, 0)),
            pl.BlockSpec((D, D), lambda i: (0, 0)),
        ],
        out_specs=pl.BlockSpec((_BN, D), lambda i: (i, 0)),
        out_shape=jax.ShapeDtypeStruct((N, D), jnp.float32),
    )(x, s, recip, uxT, uaT)


def _update_xw_body(x_ref, s_ref, r_ref, ux_ref, ua_ref, wxn_ref, h_ref, xwn_ref):
    agg = (s_ref[0] + s_ref[1]) * r_ref[...]
    h = jnp.dot(x_ref[...], ux_ref[...], preferred_element_type=jnp.float32)
    h += jnp.dot(agg, ua_ref[...], preferred_element_type=jnp.float32)
    h = jnp.maximum(h, 0.0)
    h_ref[...] = h
    xwn_ref[...] = jnp.dot(h, wxn_ref[...], preferred_element_type=jnp.float32)


def _update_xw_call(x, s, recip, uxT, uaT, wxnT):
    grid = (N // _BN,)
    return pl.pallas_call(
        _update_xw_body,
        grid=grid,
        in_specs=[
            pl.BlockSpec((_BN, D), lambda i: (i, 0)),
            pl.BlockSpec((2, _BN, D), lambda i: (0, i, 0)),
            pl.BlockSpec((_BN, 1), lambda i: (i, 0)),
            pl.BlockSpec((D, D), lambda i: (0, 0)),
            pl.BlockSpec((D, D), lambda i: (0, 0)),
            pl.BlockSpec((D, D), lambda i: (0, 0)),
        ],
        out_specs=[
            pl.BlockSpec((_BN, D), lambda i: (i, 0)),
            pl.BlockSpec((_BN, D), lambda i: (i, 0)),
        ],
        out_shape=[
            jax.ShapeDtypeStruct((N, D), jnp.float32),
            jax.ShapeDtypeStruct((N, D), jnp.float32),
        ],
    )(x, s, recip, uxT, uaT, wxnT)


def _sc_mesh():
    return plsc.VectorSubcoreMesh(core_axis_name="c", subcore_axis_name="s",
                                  num_cores=_NC, num_subcores=_NS)


def _msg_body(xw_hbm, ewb_hbm, src_hbm, dst_hbm,
              sums_hbm,
              isa, ida, isb, idb,
              gx, ge,
              acc,
              sg0, sg1, se0, se1, ss0, ss1):
    c = lax.axis_index("c")
    s = lax.axis_index("s")
    wid = s * _NC + c

    zero16 = jnp.zeros((_L,), jnp.float32)

    def _zero_row(i, _):
        for j in range(D // _L):
            gx[0, i, pl.ds(j * _L, _L)] = zero16
        return 0

    lax.fori_loop(0, _C, _zero_row, 0)

    row0 = s * _TILE_ROWS
    for k in range(-(-_TILE_ROWS // _C)):
        ln = min(_C, _TILE_ROWS - k * _C)

        @pl.when(jnp.logical_or(s < _NS - 1, k * _C + ln <= _LAST_ROWS))
        def _():
            pltpu.sync_copy(gx.at[0, pl.ds(0, ln)],
                            acc.at[pl.ds(row0 + k * _C, ln)])

    @pl.when(s == _NS - 1)
    def _():
        tail = _LAST_ROWS % _C
        if tail:
            base = 15 * _TILE_ROWS + (_LAST_ROWS // _C) * _C
            pltpu.sync_copy(gx.at[0, pl.ds(0, tail)], acc.at[pl.ds(base, tail)])

    plsc.subcore_barrier()

    base_chunk = wid * _NWCH
    gsem = (sg0, sg1)
    esem = (se0, se1)
    ssem = (ss0, ss1)
    sbuf = (isa, isb)
    dbuf = (ida, idb)

    def _load_idx(t1, kb):
        pltpu.sync_copy(src_hbm.at[pl.ds(base_chunk + t1, _SCH)], sbuf[kb])
        pltpu.sync_copy(dst_hbm.at[pl.ds(base_chunk + t1, _SCH)], dbuf[kb])

    def _issue(t1, kb, slot, st):
        pltpu.async_copy(xw_hbm.at[sbuf[kb].at[slot, 0]], gx.at[st], gsem[st])
        pltpu.async_copy(ewb_hbm.at[pl.ds((base_chunk + t1) * _C, _C)],
                         ge.at[st], esem[st])

    def _wait(t1, kb, slot, st):
        pltpu.make_async_copy(xw_hbm.at[sbuf[kb].at[slot, 0]], gx.at[st],
                              gsem[st]).wait()
        pltpu.make_async_copy(ewb_hbm.at[pl.ds((base_chunk + t1) * _C, _C)],
                              ge.at[st], esem[st]).wait()

    def _compute(st):
        def _rows(i4, _):
            for r in range(4):
                i = i4 * 4 + r
                for j in range(D // _L):
                    sl = pl.ds(j * _L, _L)
                    gx[st, i, sl] = jnp.maximum(gx[st, i, sl] + ge[st, i, sl],
                                                0.0)
            return 0

        lax.fori_loop(0, _C // 4, _rows, 0)

    def _scatter(st, kb, slot):
        pltpu.async_copy(gx.at[st], acc.at[dbuf[kb].at[slot, 0]], ssem[st],
                         add=True)

    def _wait_scatter(st, kb, slot):
        pltpu.make_async_copy(gx.at[st], acc.at[dbuf[kb].at[slot, 0]],
                              ssem[st]).wait()

    _load_idx(0, 0)
    _issue(0, 0, 0, 0)

    def _outer(t10, _):
        for k in range(10):
            t = t10 * 10 + k
            st = k % 2
            kb = (k // 5) % 2
            slot = k % 5
            nkb = ((k + 1) // 5) % 2
            nslot = (k + 1) % 5
            nst = (k + 1) % 2
            pkb = ((k - 1) // 5) % 2 if k >= 1 else ((9 // 5) % 2 if True else 0)
            pslot = (k - 1) % 5
            valid = t < _NWCH
            nvalid = t + 1 < _NWCH

            @pl.when(jnp.logical_and(t >= 1, valid))
            def _():
                _wait_scatter(nst, pkb, pslot)

            if (k + 1) % _SCH == 0:
                @pl.when(nvalid)
                def _():
                    _load_idx(t + 1, nkb)

            @pl.when(nvalid)
            def _():
                _issue(t + 1, nkb, nslot, nst)

            @pl.when(valid)
            def _():
                _wait(t, kb, slot, st)
                _compute(st)
                _scatter(st, kb, slot)
        return 0

    lax.fori_loop(0, -(-_NWCH // 10), _outer, 0)

    _wait_scatter((_NWCH - 1) % 2, ((_NWCH - 1) // 5) % 2, (_NWCH - 1) % 5)

    plsc.subcore_barrier()

    @pl.when(s < _NS - 1)
    def _():
        pltpu.sync_copy(acc.at[pl.ds(row0, _TILE_ROWS)],
                        sums_hbm.at[c, pl.ds(row0, _TILE_ROWS)])

    @pl.when(s == _NS - 1)
    def _():
        pltpu.sync_copy(acc.at[pl.ds(15 * _TILE_ROWS, _LAST_ROWS)],
                        sums_hbm.at[c, pl.ds(15 * _TILE_ROWS, _LAST_ROWS)])


def _msg_call(xw, ewb, src2, dst2):
    f = pl.kernel(
        _msg_body,
        out_type=jax.ShapeDtypeStruct((_NC, N, D), jnp.float32),
        mesh=_sc_mesh(),
        scratch_types=[
            pltpu.VMEM((_SCH, 1, _C), jnp.int32),
            pltpu.VMEM((_SCH, 1, _C), jnp.int32),
            pltpu.VMEM((_SCH, 1, _C), jnp.int32),
            pltpu.VMEM((_SCH, 1, _C), jnp.int32),
            pltpu.VMEM((2, _C, D), jnp.float32),
            pltpu.VMEM((2, _C, D), jnp.float32),
            pltpu.VMEM_SHARED((N, D), jnp.float32),
            pltpu.SemaphoreType.DMA,
            pltpu.SemaphoreType.DMA,
            pltpu.SemaphoreType.DMA,
            pltpu.SemaphoreType.DMA,
            pltpu.SemaphoreType.DMA,
            pltpu.SemaphoreType.DMA,
        ],
    )
    return f(xw, ewb, src2, dst2)


def _cnt_body(dst_hbm,
              cnts_hbm,
              ida, idb, ones_v,
              accc,
              ss0, ss1):
    c = lax.axis_index("c")
    s = lax.axis_index("s")
    wid = s * _NC + c

    zero16 = jnp.zeros((_L,), jnp.float32)
    one16 = jnp.ones((_L,), jnp.float32)

    def _fill(val):
        def _row(i, _):
            for j in range(D // _L):
                ones_v[i, pl.ds(j * _L, _L)] = val
            return 0
        lax.fori_loop(0, _CC, _row, 0)

    _fill(zero16)

    row0 = s * _TILE_ROWS
    for k in range(_TILE_ROWS // _CC):
        @pl.when(jnp.logical_or(s < _NS - 1, (k + 1) * _CC <= _LAST_ROWS))
        def _():
            pltpu.sync_copy(ones_v.at[pl.ds(0, _CC)],
                            accc.at[pl.ds(row0 + k * _CC, _CC)])

    @pl.when(s == _NS - 1)
    def _():
        tail = _LAST_ROWS % _CC
        if tail:
            base = 15 * _TILE_ROWS + (_LAST_ROWS // _CC) * _CC
            pltpu.sync_copy(ones_v.at[pl.ds(0, tail)], accc.at[pl.ds(base, tail)])

    _fill(one16)
    plsc.subcore_barrier()

    n_w = jnp.where(wid < 4, 79, 78)
    base_chunk = wid * 78 + jnp.minimum(wid, 4)
    dbuf = (ida, idb)
    ssem = (ss0, ss1)

    pltpu.sync_copy(dst_hbm.at[pl.ds(base_chunk, 1)], dbuf[0])

    def _outer(t2, _):
        for k in range(2):
            t = t2 * 2 + k
            st = k
            nst = 1 - k

            @pl.when(jnp.logical_and(t >= 1, t <= n_w))
            def _():
                pltpu.make_async_copy(ones_v, accc.at[dbuf[nst].at[0, 0]],
                                      ssem[nst]).wait()

            @pl.when(t + 1 < n_w)
            def _():
                pltpu.sync_copy(dst_hbm.at[pl.ds(base_chunk + t + 1, 1)],
                                dbuf[nst])

            @pl.when(t < n_w)
            def _():
                pltpu.async_copy(ones_v, accc.at[dbuf[st].at[0, 0]], ssem[st],
                                 add=True)
        return 0

    lax.fori_loop(0, 40, _outer, 0)

    plsc.subcore_barrier()

    @pl.when(s < _NS - 1)
    def _():
        pltpu.sync_copy(accc.at[pl.ds(row0, _TILE_ROWS)],
                        cnts_hbm.at[c, pl.ds(row0, _TILE_ROWS)])

    @pl.when(s == _NS - 1)
    def _():
        pltpu.sync_copy(accc.at[pl.ds(15 * _TILE_ROWS, _LAST_ROWS)],
                        cnts_hbm.at[c, pl.ds(15 * _TILE_ROWS, _LAST_ROWS)])


def _cnt_call(dst):
    f = pl.kernel(
        _cnt_body,
        out_type=jax.ShapeDtypeStruct((_NC, N, D), jnp.float32),
        mesh=_sc_mesh(),
        scratch_types=[
            pltpu.VMEM((1, 1, _CC), jnp.int32),
            pltpu.VMEM((1, 1, _CC), jnp.int32),
            pltpu.VMEM((_CC, D), jnp.float32),
            pltpu.VMEM_SHARED((N, D), jnp.float32),
            pltpu.SemaphoreType.DMA,
            pltpu.SemaphoreType.DMA,
        ],
    )
    return f(dst)


def kernel(x, edge_index, edge_attr, W1, b1, U1, W2, b2, U2):
    src = edge_index[0].astype(jnp.int32).reshape(_NCHUNK, 1, _C)
    dst_flat = edge_index[1].astype(jnp.int32)
    dst = dst_flat.reshape(_NCHUNK, 1, _C)

    w1xT = W1[:, :D].T
    w1eT = W1[:, D:].T
    u1xT = U1[:, :D].T
    u1aT = U1[:, D:].T
    w2xT = W2[:, :D].T
    w2eT = W2[:, D:].T
    u2xT = U2[:, :D].T
    u2aT = U2[:, D:].T

    cnt = _cnt_call(dst_flat.reshape(_NCCHUNK, 1, _CC))[:, :, 0]
    ewb1 = _ew_call(edge_attr, w1eT, b1)
    xw1 = _xw_call(x, w1xT)
    recip = (1.0 / jnp.maximum(cnt[0] + cnt[1], 1.0))[:, None]

    s1 = _msg_call(xw1, ewb1, src, dst)
    ewb2 = _ew_call(edge_attr, w2eT, b2)
    h1, xw2 = _update_xw_call(x, s1, recip, u1xT, u1aT, w2xT)

    s2 = _msg_call(xw2, ewb2, src, dst)
    h2 = _update_call(h1, s2, recip, u2xT, u2aT)
    return h2


# speedup vs baseline: 1.1662x; 1.0074x over previous
import jax
import jax.numpy as jnp
from jax import lax
from jax.experimental import pallas as pl
from jax.experimental.pallas import tpu as pltpu
from jax.experimental.pallas import tpu_sc as plsc

N = 10000
E = 320000
D = 128
DE = 16

_NC = 2
_NS = 16
_L = 16
_NW = _NC * _NS

_C = 40
_NCHUNK = E // _C
_NWCH = _NCHUNK // _NW
_SCH = 5
_CC = 128
_NCCHUNK = E // _CC
_TILE_ROWS = 640
_LAST_ROWS = N - 15 * _TILE_ROWS
_NPADC = 10240
_CPT = _NPADC // _NS
_BN = 1000
_BE = 2000


def _tile_rows(s):
    row0 = s * _TILE_ROWS
    return row0


def _ew_body(ea_ref, w_ref, b_ref, o_ref):
    a = ea_ref[...]
    o_ref[...] = jnp.dot(a, w_ref[...], preferred_element_type=jnp.float32) + b_ref[...]


def _ew_call(ea, weT, b):
    grid = (E // _BE,)
    return pl.pallas_call(
        _ew_body,
        grid=grid,
        in_specs=[
            pl.BlockSpec((_BE, DE), lambda i: (i, 0)),
            pl.BlockSpec((DE, D), lambda i: (0, 0)),
            pl.BlockSpec((1, D), lambda i: (0, 0)),
        ],
        out_specs=pl.BlockSpec((_BE, D), lambda i: (i, 0)),
        out_shape=jax.ShapeDtypeStruct((E, D), jnp.float32),
    )(ea, weT, b.reshape(1, D))


def _xw_body(x_ref, w_ref, o_ref):
    o_ref[...] = jnp.dot(x_ref[...], w_ref[...], preferred_element_type=jnp.float32)


def _xw_call(x, wxT):
    grid = (N // _BN,)
    return pl.pallas_call(
        _xw_body,
        grid=grid,
        in_specs=[
            pl.BlockSpec((_BN, D), lambda i: (i, 0)),
            pl.BlockSpec((D, D), lambda i: (0, 0)),
        ],
        out_specs=pl.BlockSpec((_BN, D), lambda i: (i, 0)),
        out_shape=jax.ShapeDtypeStruct((N, D), jnp.float32),
    )(x, wxT)


def _update_body(x_ref, s_ref, r_ref, ux_ref, ua_ref, h_ref):
    agg = (s_ref[0] + s_ref[1]) * r_ref[...]
    h = jnp.dot(x_ref[...], ux_ref[...], preferred_element_type=jnp.float32)
    h += jnp.dot(agg, ua_ref[...], preferred_element_type=jnp.float32)
    h_ref[...] = jnp.maximum(h, 0.0)


def _update_call(x, s, recip, uxT, uaT):
    grid = (N // _BN,)
    return pl.pallas_call(
        _update_body,
        grid=grid,
        in_specs=[
            pl.BlockSpec((_BN, D), lambda i: (i, 0)),
            pl.BlockSpec((2, _BN, D), lambda i: (0, i, 0)),
            pl.BlockSpec((_BN, 1), lambda i: (i, 0)),
            pl.BlockSpec((D, D), lambda i: (0, 0)),
            pl.BlockSpec((D, D), lambda i: (0, 0)),
        ],
        out_specs=pl.BlockSpec((_BN, D), lambda i: (i, 0)),
        out_shape=jax.ShapeDtypeStruct((N, D), jnp.float32),
    )(x, s, recip, uxT, uaT)


def _update_xw_body(x_ref, s_ref, r_ref, ux_ref, ua_ref, wxn_ref, h_ref, xwn_ref):
    agg = (s_ref[0] + s_ref[1]) * r_ref[...]
    h = jnp.dot(x_ref[...], ux_ref[...], preferred_element_type=jnp.float32)
    h += jnp.dot(agg, ua_ref[...], preferred_element_type=jnp.float32)
    h = jnp.maximum(h, 0.0)
    h_ref[...] = h
    xwn_ref[...] = jnp.dot(h, wxn_ref[...], preferred_element_type=jnp.float32)


def _update_xw_call(x, s, recip, uxT, uaT, wxnT):
    grid = (N // _BN,)
    return pl.pallas_call(
        _update_xw_body,
        grid=grid,
        in_specs=[
            pl.BlockSpec((_BN, D), lambda i: (i, 0)),
            pl.BlockSpec((2, _BN, D), lambda i: (0, i, 0)),
            pl.BlockSpec((_BN, 1), lambda i: (i, 0)),
            pl.BlockSpec((D, D), lambda i: (0, 0)),
            pl.BlockSpec((D, D), lambda i: (0, 0)),
            pl.BlockSpec((D, D), lambda i: (0, 0)),
        ],
        out_specs=[
            pl.BlockSpec((_BN, D), lambda i: (i, 0)),
            pl.BlockSpec((_BN, D), lambda i: (i, 0)),
        ],
        out_shape=[
            jax.ShapeDtypeStruct((N, D), jnp.float32),
            jax.ShapeDtypeStruct((N, D), jnp.float32),
        ],
    )(x, s, recip, uxT, uaT, wxnT)


def _sc_mesh():
    return plsc.VectorSubcoreMesh(core_axis_name="c", subcore_axis_name="s",
                                  num_cores=_NC, num_subcores=_NS)


def _msg_body(xw_hbm, ewb_hbm, src_hbm, dst_hbm,
              sums_hbm,
              isa, ida, isb, idb,
              gx, ge,
              acc,
              sg0, sg1, se0, se1, ss0, ss1):
    c = lax.axis_index("c")
    s = lax.axis_index("s")
    wid = s * _NC + c

    zero16 = jnp.zeros((_L,), jnp.float32)

    def _zero_row(i, _):
        for j in range(D // _L):
            gx[0, i, pl.ds(j * _L, _L)] = zero16
        return 0

    lax.fori_loop(0, _C, _zero_row, 0)

    row0 = s * _TILE_ROWS
    for k in range(-(-_TILE_ROWS // _C)):
        ln = min(_C, _TILE_ROWS - k * _C)

        @pl.when(jnp.logical_or(s < _NS - 1, k * _C + ln <= _LAST_ROWS))
        def _():
            pltpu.sync_copy(gx.at[0, pl.ds(0, ln)],
                            acc.at[pl.ds(row0 + k * _C, ln)])

    @pl.when(s == _NS - 1)
    def _():
        tail = _LAST_ROWS % _C
        if tail:
            base = 15 * _TILE_ROWS + (_LAST_ROWS // _C) * _C
            pltpu.sync_copy(gx.at[0, pl.ds(0, tail)], acc.at[pl.ds(base, tail)])

    plsc.subcore_barrier()

    base_chunk = wid * _NWCH
    gsem = (sg0, sg1)
    esem = (se0, se1)
    ssem = (ss0, ss1)
    sbuf = (isa, isb)
    dbuf = (ida, idb)

    def _load_idx(t1, kb):
        pltpu.sync_copy(src_hbm.at[pl.ds(base_chunk + t1, _SCH)], sbuf[kb])
        pltpu.sync_copy(dst_hbm.at[pl.ds(base_chunk + t1, _SCH)], dbuf[kb])

    def _issue(t1, kb, slot, st):
        pltpu.async_copy(xw_hbm.at[sbuf[kb].at[slot, 0]], gx.at[st], gsem[st])
        pltpu.async_copy(ewb_hbm.at[pl.ds((base_chunk + t1) * _C, _C)],
                         ge.at[st], esem[st])

    def _wait(t1, kb, slot, st):
        pltpu.make_async_copy(xw_hbm.at[sbuf[kb].at[slot, 0]], gx.at[st],
                              gsem[st]).wait()
        pltpu.make_async_copy(ewb_hbm.at[pl.ds((base_chunk + t1) * _C, _C)],
                              ge.at[st], esem[st]).wait()

    def _compute(st):
        def _row(i, _):
            for j in range(D // _L):
                sl = pl.ds(j * _L, _L)
                gx[st, i, sl] = jnp.maximum(gx[st, i, sl] + ge[st, i, sl], 0.0)
            return 0

        lax.fori_loop(0, _C, _row, 0)

    def _scatter(st, kb, slot):
        pltpu.async_copy(gx.at[st], acc.at[dbuf[kb].at[slot, 0]], ssem[st],
                         add=True)

    def _wait_scatter(st, kb, slot):
        pltpu.make_async_copy(gx.at[st], acc.at[dbuf[kb].at[slot, 0]],
                              ssem[st]).wait()

    _load_idx(0, 0)
    _issue(0, 0, 0, 0)

    def _outer(t10, _):
        for k in range(10):
            t = t10 * 10 + k
            st = k % 2
            kb = (k // 5) % 2
            slot = k % 5
            nkb = ((k + 1) // 5) % 2
            nslot = (k + 1) % 5
            nst = (k + 1) % 2
            pkb = ((k - 1) // 5) % 2 if k >= 1 else ((9 // 5) % 2 if True else 0)
            pslot = (k - 1) % 5
            valid = t < _NWCH
            nvalid = t + 1 < _NWCH

            @pl.when(jnp.logical_and(t >= 1, valid))
            def _():
                _wait_scatter(nst, pkb, pslot)

            if (k + 1) % _SCH == 0:
                @pl.when(nvalid)
                def _():
                    _load_idx(t + 1, nkb)

            @pl.when(nvalid)
            def _():
                _issue(t + 1, nkb, nslot, nst)

            @pl.when(valid)
            def _():
                _wait(t, kb, slot, st)
                _compute(st)
                _scatter(st, kb, slot)
        return 0

    lax.fori_loop(0, -(-_NWCH // 10), _outer, 0)

    _wait_scatter((_NWCH - 1) % 2, ((_NWCH - 1) // 5) % 2, (_NWCH - 1) % 5)

    plsc.subcore_barrier()

    @pl.when(s < _NS - 1)
    def _():
        pltpu.sync_copy(acc.at[pl.ds(row0, _TILE_ROWS)],
                        sums_hbm.at[c, pl.ds(row0, _TILE_ROWS)])

    @pl.when(s == _NS - 1)
    def _():
        pltpu.sync_copy(acc.at[pl.ds(15 * _TILE_ROWS, _LAST_ROWS)],
                        sums_hbm.at[c, pl.ds(15 * _TILE_ROWS, _LAST_ROWS)])


def _msg_call(xw, ewb, src2, dst2):
    f = pl.kernel(
        _msg_body,
        out_type=jax.ShapeDtypeStruct((_NC, N, D), jnp.float32),
        mesh=_sc_mesh(),
        scratch_types=[
            pltpu.VMEM((_SCH, 1, _C), jnp.int32),
            pltpu.VMEM((_SCH, 1, _C), jnp.int32),
            pltpu.VMEM((_SCH, 1, _C), jnp.int32),
            pltpu.VMEM((_SCH, 1, _C), jnp.int32),
            pltpu.VMEM((2, _C, D), jnp.float32),
            pltpu.VMEM((2, _C, D), jnp.float32),
            pltpu.VMEM_SHARED((N, D), jnp.float32),
            pltpu.SemaphoreType.DMA,
            pltpu.SemaphoreType.DMA,
            pltpu.SemaphoreType.DMA,
            pltpu.SemaphoreType.DMA,
            pltpu.SemaphoreType.DMA,
            pltpu.SemaphoreType.DMA,
        ],
    )
    return f(xw, ewb, src2, dst2)


def _cnt_body(dst_hbm,
              cnts_hbm,
              ida, idb, ones_v,
              accc,
              ss0, ss1):
    c = lax.axis_index("c")
    s = lax.axis_index("s")
    wid = s * _NC + c

    zero16 = jnp.zeros((_L,), jnp.float32)
    one16 = jnp.ones((_L,), jnp.float32)

    def _fill(val):
        def _row(i, _):
            for j in range(D // _L):
                ones_v[i, pl.ds(j * _L, _L)] = val
            return 0
        lax.fori_loop(0, _CC, _row, 0)

    _fill(zero16)

    row0 = s * _TILE_ROWS
    for k in range(_TILE_ROWS // _CC):
        @pl.when(jnp.logical_or(s < _NS - 1, (k + 1) * _CC <= _LAST_ROWS))
        def _():
            pltpu.sync_copy(ones_v.at[pl.ds(0, _CC)],
                            accc.at[pl.ds(row0 + k * _CC, _CC)])

    @pl.when(s == _NS - 1)
    def _():
        tail = _LAST_ROWS % _CC
        if tail:
            base = 15 * _TILE_ROWS + (_LAST_ROWS // _CC) * _CC
            pltpu.sync_copy(ones_v.at[pl.ds(0, tail)], accc.at[pl.ds(base, tail)])

    _fill(one16)
    plsc.subcore_barrier()

    n_w = jnp.where(wid < 4, 79, 78)
    base_chunk = wid * 78 + jnp.minimum(wid, 4)
    dbuf = (ida, idb)
    ssem = (ss0, ss1)

    pltpu.sync_copy(dst_hbm.at[pl.ds(base_chunk, 1)], dbuf[0])

    def _outer(t2, _):
        for k in range(2):
            t = t2 * 2 + k
            st = k
            nst = 1 - k

            @pl.when(jnp.logical_and(t >= 1, t <= n_w))
            def _():
                pltpu.make_async_copy(ones_v, accc.at[dbuf[nst].at[0, 0]],
                                      ssem[nst]).wait()

            @pl.when(t + 1 < n_w)
            def _():
                pltpu.sync_copy(dst_hbm.at[pl.ds(base_chunk + t + 1, 1)],
                                dbuf[nst])

            @pl.when(t < n_w)
            def _():
                pltpu.async_copy(ones_v, accc.at[dbuf[st].at[0, 0]], ssem[st],
                                 add=True)
        return 0

    lax.fori_loop(0, 40, _outer, 0)

    plsc.subcore_barrier()

    @pl.when(s < _NS - 1)
    def _():
        pltpu.sync_copy(accc.at[pl.ds(row0, _TILE_ROWS)],
                        cnts_hbm.at[c, pl.ds(row0, _TILE_ROWS)])

    @pl.when(s == _NS - 1)
    def _():
        pltpu.sync_copy(accc.at[pl.ds(15 * _TILE_ROWS, _LAST_ROWS)],
                        cnts_hbm.at[c, pl.ds(15 * _TILE_ROWS, _LAST_ROWS)])


def _cnt_call(dst):
    f = pl.kernel(
        _cnt_body,
        out_type=jax.ShapeDtypeStruct((_NC, N, D), jnp.float32),
        mesh=_sc_mesh(),
        scratch_types=[
            pltpu.VMEM((1, 1, _CC), jnp.int32),
            pltpu.VMEM((1, 1, _CC), jnp.int32),
            pltpu.VMEM((_CC, D), jnp.float32),
            pltpu.VMEM_SHARED((N, D), jnp.float32),
            pltpu.SemaphoreType.DMA,
            pltpu.SemaphoreType.DMA,
        ],
    )
    return f(dst)


def kernel(x, edge_index, edge_attr, W1, b1, U1, W2, b2, U2):
    src = edge_index[0].astype(jnp.int32).reshape(_NCHUNK, 1, _C)
    dst_flat = edge_index[1].astype(jnp.int32)
    dst = dst_flat.reshape(_NCHUNK, 1, _C)

    w1xT = W1[:, :D].T
    w1eT = W1[:, D:].T
    u1xT = U1[:, :D].T
    u1aT = U1[:, D:].T
    w2xT = W2[:, :D].T
    w2eT = W2[:, D:].T
    u2xT = U2[:, :D].T
    u2aT = U2[:, D:].T

    cnt = _cnt_call(dst_flat.reshape(_NCCHUNK, 1, _CC))[:, :, 0]
    ewb1 = _ew_call(edge_attr, w1eT, b1)
    xw1 = _xw_call(x, w1xT)
    recip = (1.0 / jnp.maximum(cnt[0] + cnt[1], 1.0))[:, None]

    s1 = _msg_call(xw1, ewb1, src, dst)
    ewb2 = _ew_call(edge_attr, w2eT, b2)
    h1, xw2 = _update_xw_call(x, s1, recip, u1xT, u1aT, w2xT)

    s2 = _msg_call(xw2, ewb2, src, dst)
    h2 = _update_call(h1, s2, recip, u2xT, u2aT)
    return h2
